# baseline (speedup 1.0000x reference)
import functools

import jax
import jax.numpy as jnp
from jax import lax
from jax.experimental import pallas as pl
from jax.experimental.pallas import tpu as pltpu
from jax.experimental.pallas import tpu_sc as plsc

N_INT = 10000
N_PHASE = 320000
D = 128
HID = 64

NPAD = 10240
NC = 2
NS = 16
NW = NC * NS
NROWS = N_PHASE // 128
RPW = 80
CHUNK = N_PHASE // NW

ADV_SUB = 1024
ADV_NSUB = 16
ADV_BLK = ADV_SUB * ADV_NSUB


def _adv_body(*refs):
    (*x_refs, w1_ref, b1_ref, w2_ref, o_ref) = refs
    w1 = w1_ref[...]
    b1 = b1_ref[...]
    w2 = w2_ref[...]
    for k, xr in enumerate(x_refs):
        x = xr[...]
        h = jnp.maximum(
            jnp.dot(x, w1, preferred_element_type=jnp.float32) + b1, 0.0)
        o_ref[pl.ds(k * ADV_SUB, ADV_SUB)] = jnp.dot(
            w2, h.T, preferred_element_type=jnp.float32)[0]


def _sv_body(x_ref, w1_ref, b1_ref, w2_ref, b2_ref, o_ref):
    x = x_ref[...]
    h = jnp.maximum(
        jnp.dot(x, w1_ref[...], preferred_element_type=jnp.float32) + b1_ref[...],
        0.0)
    o_ref[...] = (jnp.dot(w2_ref[...], h.T, preferred_element_type=jnp.float32)[0]
                  + b2_ref[0])


def _comb_body(s_ref, c_ref, sv_ref, o_ref):
    tot = jnp.sum(s_ref[...], axis=0)
    cnt = jnp.sum(c_ref[...], axis=0)
    o_ref[...] = sv_ref[...] - tot / jnp.maximum(cnt, 1.0)


_sc_mesh = plsc.VectorSubcoreMesh(core_axis_name="c", subcore_axis_name="s")


@functools.partial(
    pl.kernel,
    mesh=_sc_mesh,
    out_type=jax.ShapeDtypeStruct((NC, NPAD), jnp.float32),
    scratch_types=[
        pltpu.VMEM((RPW, 128), jnp.int32),
        pltpu.VMEM((RPW, 128), jnp.float32),
        pltpu.VMEM_SHARED((NPAD,), jnp.float32),
        pltpu.SemaphoreType.DMA,
    ],
)
def _segsum_k(idx_hbm, val_hbm, zeros_hbm, out_hbm, idx_v, val_v, sh_acc, sem):
    c = lax.axis_index("c")
    s = lax.axis_index("s")
    w = c * NS + s
    base = w * RPW
    nrows = jnp.where(w == NW - 1, NROWS - (NW - 1) * RPW, RPW)

    @pl.when(s == 0)
    def _():
        pltpu.sync_copy(zeros_hbm, sh_acc)

    @pl.when(w < NW - 1)
    def _():
        pltpu.sync_copy(idx_hbm.at[pl.ds(base, RPW)], idx_v)
        pltpu.sync_copy(val_hbm.at[pl.ds(base, RPW)], val_v)

    @pl.when(w == NW - 1)
    def _():
        n_last = NROWS - (NW - 1) * RPW
        pltpu.sync_copy(idx_hbm.at[pl.ds((NW - 1) * RPW, n_last)],
                        idx_v.at[pl.ds(0, n_last)])
        pltpu.sync_copy(val_hbm.at[pl.ds((NW - 1) * RPW, n_last)],
                        val_v.at[pl.ds(0, n_last)])

    plsc.subcore_barrier()

    def body(j, carry):
        pltpu.async_copy(val_v.at[j], sh_acc.at[idx_v.at[j]], sem, add=True)
        return carry

    lax.fori_loop(0, nrows, body, 0)

    def drain(j, carry):
        pltpu.make_async_copy(val_v.at[j], sh_acc.at[idx_v.at[j]], sem).wait()
        return carry

    lax.fori_loop(0, nrows, drain, 0)

    plsc.subcore_barrier()

    @pl.when(s == 0)
    def _():
        pltpu.sync_copy(sh_acc, out_hbm.at[c])


@functools.partial(
    pl.kernel,
    mesh=_sc_mesh,
    compiler_params=pltpu.CompilerParams(needs_layout_passes=False),
    out_type=jax.ShapeDtypeStruct((N_PHASE,), jnp.float32),
    scratch_types=[
        pltpu.VMEM((CHUNK,), jnp.int32),
        pltpu.VMEM((CHUNK,), jnp.float32),
        pltpu.VMEM((CHUNK,), jnp.float32),
        pltpu.VMEM((NPAD,), jnp.float32),
    ],
)
def _gather_k(idx_hbm, adv_hbm, comb_hbm, out_hbm,
              idx_v, adv_v, out_v, comb_v):
    c = lax.axis_index("c")
    s = lax.axis_index("s")
    wid = c * NS + s
    base = wid * CHUNK

    pltpu.sync_copy(idx_hbm.at[pl.ds(base, CHUNK)], idx_v)
    pltpu.sync_copy(adv_hbm.at[pl.ds(base, CHUNK)], adv_v)
    pltpu.sync_copy(comb_hbm, comb_v)

    @plsc.parallel_loop(0, CHUNK // 16, unroll=5)
    def gbody(i):
        sl = pl.ds(i * 16, 16)
        iv = idx_v[sl]
        g = plsc.load_gather(comb_v, [iv])
        out_v[sl] = adv_v[sl] + g

    pltpu.sync_copy(out_v, out_hbm.at[pl.ds(base, CHUNK)])


def kernel(intersection_x, phase_x, action_index, Ws1, bs1, Ws2, bs2,
           Wa1, ba1, Wa2, ba2):
    idx2d = action_index.reshape(NROWS, 128)
    zeros = jnp.zeros((NPAD,), jnp.float32)
    ones2d = jnp.ones((NROWS, 128), jnp.float32)

    cnts = _segsum_k(idx2d, ones2d, zeros)

    adv = pl.pallas_call(
        _adv_body,
        grid=(pl.cdiv(N_PHASE, ADV_BLK),),
        in_specs=[
            pl.BlockSpec((ADV_SUB, D),
                         lambda i, k=k: (jnp.minimum(
                             ADV_NSUB * i + k,
                             (N_PHASE - 1) // ADV_SUB), 0))
            for k in range(ADV_NSUB)
        ] + [
            pl.BlockSpec((D, HID), lambda i: (0, 0)),
            pl.BlockSpec((1, HID), lambda i: (0, 0)),
            pl.BlockSpec((1, HID), lambda i: (0, 0)),
        ],
        out_specs=pl.BlockSpec((ADV_BLK,), lambda i: (i,)),
        out_shape=jax.ShapeDtypeStruct((N_PHASE,), jnp.float32),
    )(*([phase_x] * ADV_NSUB),
      Wa1, ba1.reshape(1, HID), Wa2.reshape(1, HID))

    ixp = jnp.pad(intersection_x, ((0, NPAD - N_INT), (0, 0)))
    sv = pl.pallas_call(
        _sv_body,
        grid=(1,),
        in_specs=[
            pl.BlockSpec((NPAD, D), lambda i: (0, 0)),
            pl.BlockSpec((D, HID), lambda i: (0, 0)),
            pl.BlockSpec((1, HID), lambda i: (0, 0)),
            pl.BlockSpec((1, HID), lambda i: (0, 0)),
            pl.BlockSpec(memory_space=pltpu.SMEM),
        ],
        out_specs=pl.BlockSpec((NPAD,), lambda i: (0,)),
        out_shape=jax.ShapeDtypeStruct((NPAD,), jnp.float32),
    )(ixp, Ws1, bs1.reshape(1, HID), Ws2.reshape(1, HID), bs2)

    sums = _segsum_k(idx2d, adv.reshape(NROWS, 128), zeros)

    comb = pl.pallas_call(
        _comb_body,
        grid=(1,),
        in_specs=[
            pl.BlockSpec((NC, NPAD), lambda i: (0, 0)),
            pl.BlockSpec((NC, NPAD), lambda i: (0, 0)),
            pl.BlockSpec((NPAD,), lambda i: (0,)),
        ],
        out_specs=pl.BlockSpec((NPAD,), lambda i: (0,)),
        out_shape=jax.ShapeDtypeStruct((NPAD,), jnp.float32),
    )(sums, cnts, sv)

    out = _gather_k(action_index, adv, comb)

    return (out, action_index)


# speedup vs baseline: 1.0528x; 1.0528x over previous
import functools

import jax
import jax.numpy as jnp
from jax import lax
from jax.experimental import pallas as pl
from jax.experimental.pallas import tpu as pltpu
from jax.experimental.pallas import tpu_sc as plsc

N_INT = 10000
N_PHASE = 320000
D = 128
HID = 64

NPAD = 10240
NC = 2
NS = 16
NW = NC * NS
NROWS = N_PHASE // 128
RPW = 80
CHUNK = N_PHASE // NW

ADV_SUB = 2048
ADV_NSUB = 16
ADV_BLK = ADV_SUB * ADV_NSUB


def _adv_body(*refs):
    (*x_refs, w1_ref, b1_ref, w2_ref, o_ref) = refs
    w1 = w1_ref[...]
    b1 = b1_ref[...]
    w2 = w2_ref[...]
    for k, xr in enumerate(x_refs):
        x = xr[...]
        h = jnp.maximum(
            jnp.dot(x, w1, preferred_element_type=jnp.float32) + b1, 0.0)
        o_ref[pl.ds(k * ADV_SUB, ADV_SUB)] = jnp.dot(
            w2, h.T, preferred_element_type=jnp.float32)[0]


def _sv_body(x_ref, w1_ref, b1_ref, w2_ref, b2_ref, o_ref):
    x = x_ref[...]
    h = jnp.maximum(
        jnp.dot(x, w1_ref[...], preferred_element_type=jnp.float32) + b1_ref[...],
        0.0)
    o_ref[...] = (jnp.dot(w2_ref[...], h.T, preferred_element_type=jnp.float32)[0]
                  + b2_ref[0])


def _comb_body(s_ref, c_ref, sv_ref, o_ref):
    tot = jnp.sum(s_ref[...], axis=0)
    cnt = jnp.sum(c_ref[...], axis=0)
    o_ref[...] = sv_ref[...] - tot / jnp.maximum(cnt, 1.0)


_sc_mesh = plsc.VectorSubcoreMesh(core_axis_name="c", subcore_axis_name="s")


@functools.partial(
    pl.kernel,
    mesh=_sc_mesh,
    out_type=jax.ShapeDtypeStruct((NC, NPAD), jnp.float32),
    scratch_types=[
        pltpu.VMEM((RPW, 128), jnp.int32),
        pltpu.VMEM((RPW, 128), jnp.float32),
        pltpu.VMEM_SHARED((NPAD,), jnp.float32),
        pltpu.SemaphoreType.DMA,
    ],
)
def _segsum_k(idx_hbm, val_hbm, zeros_hbm, out_hbm, idx_v, val_v, sh_acc, sem):
    c = lax.axis_index("c")
    s = lax.axis_index("s")
    w = c * NS + s
    base = w * RPW
    nrows = jnp.where(w == NW - 1, NROWS - (NW - 1) * RPW, RPW)

    @pl.when(s == 0)
    def _():
        pltpu.sync_copy(zeros_hbm, sh_acc)

    @pl.when(w < NW - 1)
    def _():
        pltpu.sync_copy(idx_hbm.at[pl.ds(base, RPW)], idx_v)
        pltpu.sync_copy(val_hbm.at[pl.ds(base, RPW)], val_v)

    @pl.when(w == NW - 1)
    def _():
        n_last = NROWS - (NW - 1) * RPW
        pltpu.sync_copy(idx_hbm.at[pl.ds((NW - 1) * RPW, n_last)],
                        idx_v.at[pl.ds(0, n_last)])
        pltpu.sync_copy(val_hbm.at[pl.ds((NW - 1) * RPW, n_last)],
                        val_v.at[pl.ds(0, n_last)])

    plsc.subcore_barrier()

    def body(j, carry):
        pltpu.async_copy(val_v.at[j], sh_acc.at[idx_v.at[j]], sem, add=True)
        return carry

    lax.fori_loop(0, nrows, body, 0)

    def drain(j, carry):
        pltpu.make_async_copy(val_v.at[j], sh_acc.at[idx_v.at[j]], sem).wait()
        return carry

    lax.fori_loop(0, nrows, drain, 0)

    plsc.subcore_barrier()

    @pl.when(s == 0)
    def _():
        pltpu.sync_copy(sh_acc, out_hbm.at[c])


@functools.partial(
    pl.kernel,
    mesh=_sc_mesh,
    compiler_params=pltpu.CompilerParams(needs_layout_passes=False),
    out_type=jax.ShapeDtypeStruct((N_PHASE,), jnp.float32),
    scratch_types=[
        pltpu.VMEM((CHUNK,), jnp.int32),
        pltpu.VMEM((CHUNK,), jnp.float32),
        pltpu.VMEM((CHUNK,), jnp.float32),
        pltpu.VMEM((NPAD,), jnp.float32),
    ],
)
def _gather_k(idx_hbm, adv_hbm, comb_hbm, out_hbm,
              idx_v, adv_v, out_v, comb_v):
    c = lax.axis_index("c")
    s = lax.axis_index("s")
    wid = c * NS + s
    base = wid * CHUNK

    pltpu.sync_copy(idx_hbm.at[pl.ds(base, CHUNK)], idx_v)
    pltpu.sync_copy(adv_hbm.at[pl.ds(base, CHUNK)], adv_v)
    pltpu.sync_copy(comb_hbm, comb_v)

    @plsc.parallel_loop(0, CHUNK // 16, unroll=5)
    def gbody(i):
        sl = pl.ds(i * 16, 16)
        iv = idx_v[sl]
        g = plsc.load_gather(comb_v, [iv])
        out_v[sl] = adv_v[sl] + g

    pltpu.sync_copy(out_v, out_hbm.at[pl.ds(base, CHUNK)])


def kernel(intersection_x, phase_x, action_index, Ws1, bs1, Ws2, bs2,
           Wa1, ba1, Wa2, ba2):
    idx2d = action_index.reshape(NROWS, 128)
    zeros = jnp.zeros((NPAD,), jnp.float32)
    ones2d = jnp.ones((NROWS, 128), jnp.float32)

    cnts = _segsum_k(idx2d, ones2d, zeros)

    adv = pl.pallas_call(
        _adv_body,
        grid=(pl.cdiv(N_PHASE, ADV_BLK),),
        in_specs=[
            pl.BlockSpec((ADV_SUB, D),
                         lambda i, k=k: (jnp.minimum(
                             ADV_NSUB * i + k,
                             (N_PHASE - 1) // ADV_SUB), 0))
            for k in range(ADV_NSUB)
        ] + [
            pl.BlockSpec((D, HID), lambda i: (0, 0)),
            pl.BlockSpec((1, HID), lambda i: (0, 0)),
            pl.BlockSpec((1, HID), lambda i: (0, 0)),
        ],
        out_specs=pl.BlockSpec((ADV_BLK,), lambda i: (i,)),
        out_shape=jax.ShapeDtypeStruct((N_PHASE,), jnp.float32),
    )(*([phase_x] * ADV_NSUB),
      Wa1, ba1.reshape(1, HID), Wa2.reshape(1, HID))

    ixp = jnp.pad(intersection_x, ((0, NPAD - N_INT), (0, 0)))
    sv = pl.pallas_call(
        _sv_body,
        grid=(1,),
        in_specs=[
            pl.BlockSpec((NPAD, D), lambda i: (0, 0)),
            pl.BlockSpec((D, HID), lambda i: (0, 0)),
            pl.BlockSpec((1, HID), lambda i: (0, 0)),
            pl.BlockSpec((1, HID), lambda i: (0, 0)),
            pl.BlockSpec(memory_space=pltpu.SMEM),
        ],
        out_specs=pl.BlockSpec((NPAD,), lambda i: (0,)),
        out_shape=jax.ShapeDtypeStruct((NPAD,), jnp.float32),
    )(ixp, Ws1, bs1.reshape(1, HID), Ws2.reshape(1, HID), bs2)

    sums = _segsum_k(idx2d, adv.reshape(NROWS, 128), zeros)

    comb = pl.pallas_call(
        _comb_body,
        grid=(1,),
        in_specs=[
            pl.BlockSpec((NC, NPAD), lambda i: (0, 0)),
            pl.BlockSpec((NC, NPAD), lambda i: (0, 0)),
            pl.BlockSpec((NPAD,), lambda i: (0,)),
        ],
        out_specs=pl.BlockSpec((NPAD,), lambda i: (0,)),
        out_shape=jax.ShapeDtypeStruct((NPAD,), jnp.float32),
    )(sums, cnts, sv)

    out = _gather_k(action_index, adv, comb)

    return (out, action_index)
